# Initial kernel scaffold; baseline (speedup 1.0000x reference)
import jax
import jax.numpy as jnp
from jax.experimental import pallas as pl


def kernel(x, codebook):
    raise NotImplementedError("write your pallas kernel here")


# speedup vs baseline: 1.6895x; 1.6895x over previous
import functools

import jax
import jax.numpy as jnp
from jax.experimental import pallas as pl

_BLK = 1024


def _vq_block_kernel(x_ref, cb_ref, ek_ref, ids_ref):
    x = x_ref[...]
    cb = cb_ref[...]
    scores = jax.lax.dot_general(
        x, cb, (((1,), (1,)), ((), ())), preferred_element_type=jnp.float32
    )
    x2 = jnp.sum(x * x, axis=1, keepdims=True)
    c2 = jnp.sum(cb * cb, axis=1)[None, :]
    dist = x2 - 2.0 * scores + c2
    ids = jnp.argmin(dist, axis=1).astype(jnp.int32)
    onehot = (
        jax.lax.broadcasted_iota(jnp.int32, dist.shape, 1) == ids[:, None]
    ).astype(jnp.float32)
    ek = jax.lax.dot_general(
        onehot, cb, (((1,), (0,)), ((), ())), preferred_element_type=jnp.float32
    )
    ek_ref[...] = ek
    ids_ref[...] = ids


@functools.partial(jax.jit, static_argnames=())
def kernel(x, codebook):
    n, d = x.shape
    k = codebook.shape[0]
    grid = (n // _BLK,)
    ek, ids = pl.pallas_call(
        _vq_block_kernel,
        grid=grid,
        in_specs=[
            pl.BlockSpec((_BLK, d), lambda i: (i, 0)),
            pl.BlockSpec((k, d), lambda i: (0, 0)),
        ],
        out_specs=[
            pl.BlockSpec((_BLK, d), lambda i: (i, 0)),
            pl.BlockSpec((_BLK,), lambda i: (i,)),
        ],
        out_shape=[
            jax.ShapeDtypeStruct((n, d), jnp.float32),
            jax.ShapeDtypeStruct((n,), jnp.int32),
        ],
    )(x, codebook)
    return (ek, ids)


# speedup vs baseline: 1.8585x; 1.1000x over previous
import functools

import jax
import jax.numpy as jnp
from jax.experimental import pallas as pl
from jax.experimental.pallas import tpu as pltpu

_BLK = 1024


def _vq_block_kernel(x_ref, cb_ref, ek_ref, ids_ref, c2_ref):
    @pl.when(pl.program_id(0) == 0)
    def _init():
        cbi = cb_ref[...]
        c2_ref[...] = jnp.sum(cbi * cbi, axis=1, keepdims=True)

    x = x_ref[...]
    cb = cb_ref[...]
    k = cb.shape[0]
    scoresT = jax.lax.dot_general(
        cb, x, (((1,), (1,)), ((), ())), preferred_element_type=jnp.float32
    )
    distT = c2_ref[...] - 2.0 * scoresT
    code_iota = jax.lax.broadcasted_iota(jnp.int32, distT.shape, 0)
    mind = jnp.min(distT, axis=0)
    ids = jnp.min(
        jnp.where(distT <= mind[None, :], code_iota, k), axis=0
    ).astype(jnp.int32)
    onehotT = (code_iota == ids[None, :]).astype(jnp.float32)
    ek = jax.lax.dot_general(
        onehotT, cb, (((0,), (0,)), ((), ())), preferred_element_type=jnp.float32
    )
    ek_ref[...] = ek
    ids_ref[...] = ids


@functools.partial(jax.jit, static_argnames=())
def kernel(x, codebook):
    n, d = x.shape
    k = codebook.shape[0]
    grid = (n // _BLK,)
    ek, ids = pl.pallas_call(
        _vq_block_kernel,
        grid=grid,
        in_specs=[
            pl.BlockSpec((_BLK, d), lambda i: (i, 0)),
            pl.BlockSpec((k, d), lambda i: (0, 0)),
        ],
        out_specs=[
            pl.BlockSpec((_BLK, d), lambda i: (i, 0)),
            pl.BlockSpec((_BLK,), lambda i: (i,)),
        ],
        out_shape=[
            jax.ShapeDtypeStruct((n, d), jnp.float32),
            jax.ShapeDtypeStruct((n,), jnp.int32),
        ],
        scratch_shapes=[pltpu.VMEM((k, 1), jnp.float32)],
    )(x, codebook)
    return (ek, ids)
